# baseline (speedup 1.0000x reference)
import functools
import jax
import jax.numpy as jnp
from jax import lax
from jax.experimental import pallas as pl
from jax.experimental.pallas import tpu as pltpu
from jax.experimental.pallas import tpu_sc as plsc

ROWS, COLS = 320000, 128
NC, NS = 2, 16
NW = NC * NS
RPW = ROWS // NW
CH = 400
NCHUNK = RPW // CH
NG = COLS // 16


def _sc_body(m_hbm, out_hbm, buf0, buf1, accv, gbuf, shared, sem0, sem1):
    cid = lax.axis_index("c")
    sid = lax.axis_index("s")
    wid = sid * NC + cid
    base = wid * RPW

    bufs = (buf0, buf1)
    sems = (sem0, sem1)

    accs = tuple(jnp.full((16,), -jnp.inf, jnp.float32) for _ in range(NG))

    def row_body(buf):
        def body(r, accs):
            return tuple(
                jnp.maximum(accs[g], buf[r, pl.ds(g * 16, 16)]) for g in range(NG)
            )
        return body

    cp = pltpu.async_copy(m_hbm.at[pl.ds(base, CH), :], buf0, sem0)
    for k in range(NCHUNK):
        cur = bufs[k % 2]
        nxt_cp = None
        if k + 1 < NCHUNK:
            nxt = bufs[(k + 1) % 2]
            nxt_cp = pltpu.async_copy(
                m_hbm.at[pl.ds(base + (k + 1) * CH, CH), :], nxt, sems[(k + 1) % 2]
            )
        cp.wait()
        accs = lax.fori_loop(0, CH, row_body(cur), accs)
        cp = nxt_cp

    for g in range(NG):
        accv[pl.ds(g * 16, 16)] = accs[g]
    pltpu.sync_copy(accv, shared.at[sid])
    plsc.subcore_barrier()

    @pl.when(sid == 0)
    def _():
        pltpu.sync_copy(shared, gbuf)
        red = tuple(gbuf[0, pl.ds(g * 16, 16)] for g in range(NG))
        for s in range(1, NS):
            red = tuple(
                jnp.maximum(red[g], gbuf[s, pl.ds(g * 16, 16)]) for g in range(NG)
            )
        for g in range(NG):
            accv[pl.ds(g * 16, 16)] = red[g]
        pltpu.sync_copy(accv, out_hbm.at[cid])


def _combine_body(p_ref, o_ref):
    o_ref[...] = jnp.max(p_ref[...], axis=0, keepdims=True)


def kernel(M):
    mesh = plsc.VectorSubcoreMesh(core_axis_name="c", subcore_axis_name="s")
    sc_call = pl.kernel(
        _sc_body,
        mesh=mesh,
        out_type=jax.ShapeDtypeStruct((NC, COLS), jnp.float32),
        scratch_types=[
            pltpu.VMEM((CH, COLS), jnp.float32),
            pltpu.VMEM((CH, COLS), jnp.float32),
            pltpu.VMEM((COLS,), jnp.float32),
            pltpu.VMEM((NS, COLS), jnp.float32),
            pltpu.VMEM_SHARED((NS, COLS), jnp.float32),
            pltpu.SemaphoreType.DMA,
            pltpu.SemaphoreType.DMA,
        ],
    )
    partials = sc_call(M)
    out = pl.pallas_call(
        _combine_body,
        out_shape=jax.ShapeDtypeStruct((1, COLS), jnp.float32),
    )(partials)
    return out[0]


# speedup vs baseline: 1.7135x; 1.7135x over previous
import jax
import jax.numpy as jnp
from jax.experimental import pallas as pl
from jax.experimental.pallas import tpu as pltpu

ROWS, COLS = 320000, 128
CH = 2000
NSTEP = ROWS // CH
NBUF = 8
NSUB = 5
SUBV = CH // 8 // NSUB


def _chunk_max(buf):
    x3 = buf[...].reshape(CH // 8, 8, COLS)
    parts = [
        jnp.max(x3[i * SUBV:(i + 1) * SUBV], axis=0) for i in range(NSUB)
    ]
    p01 = jnp.maximum(parts[0], parts[1])
    p23 = jnp.maximum(parts[2], parts[3])
    return jnp.maximum(p01, jnp.maximum(p23, parts[4]))


def _max_pipelined(m_hbm, o_ref, acc, *rest):
    i = pl.program_id(0)
    bufs = tuple(rest[:NBUF])
    sems = tuple(rest[NBUF:])

    @pl.when(i == 0)
    def _prime():
        acc[...] = jnp.full_like(acc, -jnp.inf)
        for b in range(NBUF):
            pltpu.make_async_copy(
                m_hbm.at[pl.ds(b * CH, CH), :], bufs[b], sems[b]
            ).start()

    for b in range(NBUF):
        @pl.when(jax.lax.rem(i, NBUF) == b)
        def _step(b=b):
            pltpu.make_async_copy(
                m_hbm.at[pl.ds(i * CH, CH), :], bufs[b], sems[b]
            ).wait()
            acc[...] = jnp.maximum(acc[...], _chunk_max(bufs[b]))

            @pl.when(i + NBUF < NSTEP)
            def _next():
                pltpu.make_async_copy(
                    m_hbm.at[pl.ds((i + NBUF) * CH, CH), :], bufs[b], sems[b]
                ).start()

    @pl.when(i == NSTEP - 1)
    def _fin():
        o_ref[...] = jnp.max(acc[...], axis=0, keepdims=True)


def kernel(M):
    out = pl.pallas_call(
        _max_pipelined,
        grid=(NSTEP,),
        in_specs=[pl.BlockSpec(memory_space=pl.ANY)],
        out_specs=pl.BlockSpec(memory_space=pltpu.VMEM),
        out_shape=jax.ShapeDtypeStruct((1, COLS), jnp.float32),
        scratch_shapes=[pltpu.VMEM((8, COLS), jnp.float32)]
        + [pltpu.VMEM((CH, COLS), jnp.float32) for _ in range(NBUF)]
        + [pltpu.SemaphoreType.DMA for _ in range(NBUF)],
    )(M)
    return out[0]


# speedup vs baseline: 1.7793x; 1.0384x over previous
import jax
import jax.numpy as jnp
from jax.experimental import pallas as pl
from jax.experimental.pallas import tpu as pltpu

ROWS, COLS = 320000, 128
CH = 4000
NSTEP = ROWS // CH
NBUF = 6
NSUB = 5
SUBV = CH // 8 // NSUB


def _chunk_max(buf):
    x3 = buf[...].reshape(CH // 8, 8, COLS)
    parts = [
        jnp.max(x3[i * SUBV:(i + 1) * SUBV], axis=0) for i in range(NSUB)
    ]
    p01 = jnp.maximum(parts[0], parts[1])
    p23 = jnp.maximum(parts[2], parts[3])
    return jnp.maximum(p01, jnp.maximum(p23, parts[4]))


def _max_pipelined(m_hbm, o_ref, acc, *rest):
    i = pl.program_id(0)
    bufs = tuple(rest[:NBUF])
    sems = tuple(rest[NBUF:])

    @pl.when(i == 0)
    def _prime():
        acc[...] = jnp.full_like(acc, -jnp.inf)
        for b in range(NBUF):
            pltpu.make_async_copy(
                m_hbm.at[pl.ds(b * CH, CH), :], bufs[b], sems[b]
            ).start()

    for b in range(NBUF):
        @pl.when(jax.lax.rem(i, NBUF) == b)
        def _step(b=b):
            pltpu.make_async_copy(
                m_hbm.at[pl.ds(i * CH, CH), :], bufs[b], sems[b]
            ).wait()
            acc[...] = jnp.maximum(acc[...], _chunk_max(bufs[b]))

            @pl.when(i + NBUF < NSTEP)
            def _next():
                pltpu.make_async_copy(
                    m_hbm.at[pl.ds((i + NBUF) * CH, CH), :], bufs[b], sems[b]
                ).start()

    @pl.when(i == NSTEP - 1)
    def _fin():
        o_ref[...] = jnp.max(acc[...], axis=0, keepdims=True)


def kernel(M):
    out = pl.pallas_call(
        _max_pipelined,
        grid=(NSTEP,),
        in_specs=[pl.BlockSpec(memory_space=pl.ANY)],
        out_specs=pl.BlockSpec(memory_space=pltpu.VMEM),
        out_shape=jax.ShapeDtypeStruct((1, COLS), jnp.float32),
        scratch_shapes=[pltpu.VMEM((8, COLS), jnp.float32)]
        + [pltpu.VMEM((CH, COLS), jnp.float32) for _ in range(NBUF)]
        + [pltpu.SemaphoreType.DMA for _ in range(NBUF)],
    )(M)
    return out[0]
